# baseline (speedup 1.0000x reference)
import functools

import jax
import jax.numpy as jnp
from jax import lax
from jax.experimental import pallas as pl
from jax.experimental.pallas import tpu as pltpu
from jax.experimental.pallas import tpu_sc as plsc

B, L, D, V = 4096, 200, 64, 1024
N = B * L
NC, NS = 2, 16
NW = NC * NS
PER_W = N // NW
C = 128
NIT = PER_W // C

_mesh = plsc.VectorSubcoreMesh(core_axis_name="c", subcore_axis_name="s")


@functools.partial(
    pl.kernel,
    mesh=_mesh,
    compiler_params=pltpu.CompilerParams(use_tc_tiling_on_sc=False),
    out_type=jax.ShapeDtypeStruct((N, D), jnp.float32),
    scratch_types=[
        pltpu.VMEM((C,), jnp.int32),
        pltpu.VMEM((C,), jnp.int32),
        pltpu.VMEM((C, D), jnp.float32),
        pltpu.VMEM((C, D), jnp.float32),
        pltpu.SemaphoreType.DMA,
        pltpu.SemaphoreType.DMA,
    ],
)
def _emb_sum_kernel(i1_hbm, i2_hbm, t1_hbm, t2_hbm, out_hbm,
                    idx1, idx2, r1, r2, sem1, sem2):
    wid = lax.axis_index("s") * NC + lax.axis_index("c")
    base = wid * PER_W

    def body(it, carry):
        off = base + it * C
        pltpu.sync_copy(i1_hbm.at[pl.ds(off, C)], idx1)
        pltpu.sync_copy(i2_hbm.at[pl.ds(off, C)], idx2)
        cp1 = pltpu.async_copy(t1_hbm.at[idx1], r1, sem1)
        cp1.wait()
        pltpu.sync_copy(t2_hbm.at[idx2], r1, add=True)
        pltpu.sync_copy(r1, out_hbm.at[pl.ds(off, C)])
        return carry

    lax.fori_loop(0, NIT, body, 0)


def kernel(pos_1, pos_2, table1, table2):
    zrow = jnp.zeros((1, D), jnp.float32)
    t1a = jnp.concatenate([zrow, table1[: V - 1]], axis=0)
    t2a = jnp.concatenate([zrow, table2[: V - 1]], axis=0)
    i1 = pos_1.reshape(N).astype(jnp.int32)
    i2 = pos_2.reshape(N).astype(jnp.int32)
    out = _emb_sum_kernel(i1, i2, t1a, t2a)
    return out.reshape(B, L, D)


# speedup vs baseline: 1.3360x; 1.3360x over previous
import functools

import jax
import jax.numpy as jnp
from jax import lax
from jax.experimental import pallas as pl
from jax.experimental.pallas import tpu as pltpu
from jax.experimental.pallas import tpu_sc as plsc

B, L, D, V = 4096, 200, 64, 1024
N = B * L
NC, NS = 2, 16
NW = NC * NS
PER_W = N // NW
C = 128
NBUF = 8
GROUP = NBUF * C
NGRP = PER_W // GROUP

_mesh = plsc.VectorSubcoreMesh(core_axis_name="c", subcore_axis_name="s")


@functools.partial(
    pl.kernel,
    mesh=_mesh,
    compiler_params=pltpu.CompilerParams(use_tc_tiling_on_sc=False),
    out_type=jax.ShapeDtypeStruct((N, D), jnp.float32),
    scratch_types=[
        pltpu.VMEM((PER_W,), jnp.int32),
        pltpu.VMEM((PER_W,), jnp.int32),
        pltpu.VMEM((NBUF, C, D), jnp.float32),
        pltpu.SemaphoreType.DMA,
        pltpu.SemaphoreType.DMA,
        pltpu.SemaphoreType.DMA,
    ],
)
def _emb_sum_kernel(i1_hbm, i2_hbm, t1_hbm, t2_hbm, out_hbm,
                    idx1, idx2, rbuf, sem1, sem2, sem3):
    wid = lax.axis_index("s") * NC + lax.axis_index("c")
    base = wid * PER_W
    pltpu.sync_copy(i1_hbm.at[pl.ds(base, PER_W)], idx1)
    pltpu.sync_copy(i2_hbm.at[pl.ds(base, PER_W)], idx2)

    def group(g, carry):
        goff = g * GROUP
        g1 = [pltpu.async_copy(
                  t1_hbm.at[idx1.at[pl.ds(goff + b * C, C)]],
                  rbuf.at[b], sem1)
              for b in range(NBUF)]
        for b in range(NBUF):
            g1[b].wait()
        g2 = [pltpu.async_copy(
                  t2_hbm.at[idx2.at[pl.ds(goff + b * C, C)]],
                  rbuf.at[b], sem2, add=True)
              for b in range(NBUF)]
        for b in range(NBUF):
            g2[b].wait()
        st = [pltpu.async_copy(
                  rbuf.at[b],
                  out_hbm.at[pl.ds(base + goff + b * C, C)], sem3)
              for b in range(NBUF)]
        for b in range(NBUF):
            st[b].wait()
        return carry

    lax.fori_loop(0, NGRP, group, 0)


def kernel(pos_1, pos_2, table1, table2):
    zrow = jnp.zeros((1, D), jnp.float32)
    t1a = jnp.concatenate([zrow, table1[: V - 1]], axis=0)
    t2a = jnp.concatenate([zrow, table2[: V - 1]], axis=0)
    i1 = pos_1.reshape(N).astype(jnp.int32)
    i2 = pos_2.reshape(N).astype(jnp.int32)
    out = _emb_sum_kernel(i1, i2, t1a, t2a)
    return out.reshape(B, L, D)


# speedup vs baseline: 1.3386x; 1.0019x over previous
import functools

import jax
import jax.numpy as jnp
from jax import lax
from jax.experimental import pallas as pl
from jax.experimental.pallas import tpu as pltpu
from jax.experimental.pallas import tpu_sc as plsc

B, L, D, V = 4096, 200, 64, 1024
N = B * L
NC, NS = 2, 16
NW = NC * NS
BPW = B // NW
PER_W = BPW * L
C1, C2 = 128, L - 128
NBUF = 4
NGRP = BPW // NBUF

_mesh = plsc.VectorSubcoreMesh(core_axis_name="c", subcore_axis_name="s")


@functools.partial(
    pl.kernel,
    mesh=_mesh,
    compiler_params=pltpu.CompilerParams(use_tc_tiling_on_sc=False),
    out_type=jax.ShapeDtypeStruct((B, L, D), jnp.float32),
    scratch_types=[
        pltpu.VMEM((PER_W,), jnp.int32),
        pltpu.VMEM((PER_W,), jnp.int32),
        pltpu.VMEM((NBUF, L, D), jnp.float32),
        pltpu.SemaphoreType.DMA,
        pltpu.SemaphoreType.DMA,
        pltpu.SemaphoreType.DMA,
    ],
)
def _emb_sum_kernel(i1_hbm, i2_hbm, t1_hbm, t2_hbm, out_hbm,
                    idx1, idx2, rbuf, sem1, sem2, sem3):
    wid = lax.axis_index("s") * NC + lax.axis_index("c")
    base = wid * PER_W
    pltpu.sync_copy(i1_hbm.at[pl.ds(base, PER_W)], idx1)
    pltpu.sync_copy(i2_hbm.at[pl.ds(base, PER_W)], idx2)

    def gathers(table, idx, g, sem, add):
        cps = []
        for b in range(NBUF):
            goff = (g * NBUF + b) * L
            cps.append(pltpu.async_copy(
                table.at[idx.at[pl.ds(goff, C1)]],
                rbuf.at[b, pl.ds(0, C1)], sem, add=add))
            cps.append(pltpu.async_copy(
                table.at[idx.at[pl.ds(goff + C1, C2)]],
                rbuf.at[b, pl.ds(C1, C2)], sem, add=add))
        return cps

    def group(g, carry):
        for cp in gathers(t1_hbm, idx1, g, sem1, False):
            cp.wait()
        for cp in gathers(t2_hbm, idx2, g, sem2, True):
            cp.wait()
        st = [pltpu.async_copy(
                  rbuf.at[b],
                  out_hbm.at[wid * BPW + g * NBUF + b], sem3)
              for b in range(NBUF)]
        for cp in st:
            cp.wait()
        return carry

    lax.fori_loop(0, NGRP, group, 0)


def kernel(pos_1, pos_2, table1, table2):
    zrow = jnp.zeros((1, D), jnp.float32)
    t1a = jnp.concatenate([zrow, table1[: V - 1]], axis=0)
    t2a = jnp.concatenate([zrow, table2[: V - 1]], axis=0)
    i1 = pos_1.reshape(N).astype(jnp.int32)
    i2 = pos_2.reshape(N).astype(jnp.int32)
    return _emb_sum_kernel(i1, i2, t1a, t2a)


# speedup vs baseline: 1.9792x; 1.4786x over previous
import functools

import jax
import jax.numpy as jnp
from jax import lax
from jax.experimental import pallas as pl
from jax.experimental.pallas import tpu as pltpu
from jax.experimental.pallas import tpu_sc as plsc

B, L, D, V = 4096, 200, 64, 1024
N = B * L
PW = 128
NC, NS = 2, 16
NW = NC * NS
BPW = B // NW
PER_W = BPW * L
C1, C2 = 128, L - 128
NBUF = 4
NGRP = BPW // NBUF

_mesh = plsc.VectorSubcoreMesh(core_axis_name="c", subcore_axis_name="s")


@functools.partial(
    pl.kernel,
    mesh=_mesh,
    compiler_params=pltpu.CompilerParams(use_tc_tiling_on_sc=False),
    out_type=jax.ShapeDtypeStruct((B, L, PW), jnp.float32),
    scratch_types=[
        pltpu.VMEM((PER_W,), jnp.int32),
        pltpu.VMEM((PER_W,), jnp.int32),
        pltpu.VMEM((NBUF, L, D), jnp.float32),
        pltpu.SemaphoreType.DMA,
        pltpu.SemaphoreType.DMA,
        pltpu.SemaphoreType.DMA,
    ],
)
def _emb_sum_kernel(i1_hbm, i2_hbm, t1_hbm, t2_hbm, out_hbm,
                    idx1, idx2, rbuf, sem1, sem2, sem3):
    wid = lax.axis_index("s") * NC + lax.axis_index("c")
    base = wid * PER_W
    pltpu.sync_copy(i1_hbm.at[pl.ds(base, PER_W)], idx1)
    pltpu.sync_copy(i2_hbm.at[pl.ds(base, PER_W)], idx2)

    def gathers(table, idx, g, sem, add):
        cps = []
        for b in range(NBUF):
            goff = (g * NBUF + b) * L
            cps.append(pltpu.async_copy(
                table.at[idx.at[pl.ds(goff, C1)]],
                rbuf.at[b, pl.ds(0, C1)], sem, add=add))
            cps.append(pltpu.async_copy(
                table.at[idx.at[pl.ds(goff + C1, C2)]],
                rbuf.at[b, pl.ds(C1, C2)], sem, add=add))
        return cps

    def group(g, carry):
        for cp in gathers(t1_hbm, idx1, g, sem1, False):
            cp.wait()
        for cp in gathers(t2_hbm, idx2, g, sem2, True):
            cp.wait()
        st = [pltpu.async_copy(
                  rbuf.at[b],
                  out_hbm.at[wid * BPW + g * NBUF + b, :, pl.ds(0, D)], sem3)
              for b in range(NBUF)]
        for cp in st:
            cp.wait()
        return carry

    lax.fori_loop(0, NGRP, group, 0)


def kernel(pos_1, pos_2, table1, table2):
    zrow = jnp.zeros((1, D), jnp.float32)
    t1a = jnp.concatenate([zrow, table1[: V - 1]], axis=0)
    t2a = jnp.concatenate([zrow, table2[: V - 1]], axis=0)
    i1 = pos_1.reshape(N).astype(jnp.int32)
    i2 = pos_2.reshape(N).astype(jnp.int32)
    out = _emb_sum_kernel(i1, i2, t1a, t2a)
    return out[..., :D]


# speedup vs baseline: 1.9850x; 1.0029x over previous
import functools

import jax
import jax.numpy as jnp
from jax import lax
from jax.experimental import pallas as pl
from jax.experimental.pallas import tpu as pltpu
from jax.experimental.pallas import tpu_sc as plsc

B, L, D, V = 4096, 200, 64, 1024
N = B * L
PW = 128
NC, NS = 2, 16
NW = NC * NS
BPW = B // NW
PER_W = BPW * L
C1, C2 = 128, L - 128
NBUF = 4
NGRP = BPW // NBUF

_mesh = plsc.VectorSubcoreMesh(core_axis_name="c", subcore_axis_name="s")


@functools.partial(
    pl.kernel,
    mesh=_mesh,
    compiler_params=pltpu.CompilerParams(use_tc_tiling_on_sc=False),
    out_type=jax.ShapeDtypeStruct((B, L, PW), jnp.float32),
    scratch_types=[
        pltpu.VMEM((PER_W,), jnp.int32),
        pltpu.VMEM((PER_W,), jnp.int32),
        pltpu.VMEM((NBUF, L, D), jnp.float32),
        pltpu.SemaphoreType.DMA((NBUF,)),
        pltpu.SemaphoreType.DMA((NBUF,)),
        pltpu.SemaphoreType.DMA((NBUF,)),
    ],
)
def _emb_sum_kernel(i1_hbm, i2_hbm, t1_hbm, t2_hbm, out_hbm,
                    idx1, idx2, rbuf, sem1, sem2, sem3):
    wid = lax.axis_index("s") * NC + lax.axis_index("c")
    base = wid * PER_W
    pltpu.sync_copy(i1_hbm.at[pl.ds(base, PER_W)], idx1)
    pltpu.sync_copy(i2_hbm.at[pl.ds(base, PER_W)], idx2)

    def gather_parts(table, idx, g, b):
        goff = (g * NBUF + b) * L
        return [
            (table.at[idx.at[pl.ds(goff, C1)]], rbuf.at[b, pl.ds(0, C1)]),
            (table.at[idx.at[pl.ds(goff + C1, C2)]], rbuf.at[b, pl.ds(C1, C2)]),
        ]

    def fire_gathers(table, idx, g, b, sem, add):
        for src, dst in gather_parts(table, idx, g, b):
            pltpu.async_copy(src, dst, sem.at[b], add=add)

    def wait_gathers(table, idx, g, b, sem):
        for src, dst in gather_parts(table, idx, g, b):
            pltpu.make_async_copy(src, dst, sem.at[b]).wait()

    def store_pair(g, b):
        return (rbuf.at[b],
                out_hbm.at[wid * BPW + g * NBUF + b, :, pl.ds(0, D)])

    def group(g, carry):
        for b in range(NBUF):
            @pl.when(g > 0)
            def _drain(b=b):
                src, dst = store_pair(g, b)
                pltpu.make_async_copy(src, dst, sem3.at[b]).wait()
            fire_gathers(t1_hbm, idx1, g, b, sem1, False)
        for b in range(NBUF):
            wait_gathers(t1_hbm, idx1, g, b, sem1)
            fire_gathers(t2_hbm, idx2, g, b, sem2, True)
        for b in range(NBUF):
            wait_gathers(t2_hbm, idx2, g, b, sem2)
            src, dst = store_pair(g, b)
            pltpu.async_copy(src, dst, sem3.at[b])
        return carry

    lax.fori_loop(0, NGRP, group, 0)

    for b in range(NBUF):
        src, dst = store_pair(NGRP - 1, b)
        pltpu.make_async_copy(src, dst, sem3.at[b]).wait()


def kernel(pos_1, pos_2, table1, table2):
    zrow = jnp.zeros((1, D), jnp.float32)
    t1a = jnp.concatenate([zrow, table1[: V - 1]], axis=0)
    t2a = jnp.concatenate([zrow, table2[: V - 1]], axis=0)
    i1 = pos_1.reshape(N).astype(jnp.int32)
    i2 = pos_2.reshape(N).astype(jnp.int32)
    out = _emb_sum_kernel(i1, i2, t1a, t2a)
    return out[..., :D]


# speedup vs baseline: 3.0340x; 1.5285x over previous
import functools

import jax
import jax.numpy as jnp
from jax import lax
from jax.experimental import pallas as pl
from jax.experimental.pallas import tpu as pltpu
from jax.experimental.pallas import tpu_sc as plsc

B, L, D, V = 4096, 200, 64, 1024
N = B * L
PW = 128
NC, NS = 2, 16
NW = NC * NS
BPW = B // NW
PER_W = BPW * L
C1, C2 = 128, L - 128
NBUF = 4
NGRP = BPW // NBUF

_mesh = plsc.VectorSubcoreMesh(core_axis_name="c", subcore_axis_name="s")


@functools.partial(
    pl.kernel,
    mesh=_mesh,
    compiler_params=pltpu.CompilerParams(use_tc_tiling_on_sc=False),
    out_type=jax.ShapeDtypeStruct((B, L, PW), jnp.float32),
    scratch_types=[
        pltpu.VMEM((PER_W,), jnp.int32),
        pltpu.VMEM((PER_W,), jnp.int32),
        pltpu.VMEM((NBUF, L, D), jnp.float32),
        pltpu.VMEM_SHARED((V, D), jnp.float32),
        pltpu.VMEM_SHARED((V, D), jnp.float32),
        pltpu.SemaphoreType.DMA((NBUF,)),
        pltpu.SemaphoreType.DMA((NBUF,)),
        pltpu.SemaphoreType.DMA((NBUF,)),
    ],
)
def _emb_sum_kernel(i1_hbm, i2_hbm, t1_hbm, t2_hbm, out_hbm,
                    idx1, idx2, rbuf, sh1, sh2, sem1, sem2, sem3):
    wid = lax.axis_index("s") * NC + lax.axis_index("c")
    base = wid * PER_W

    @pl.when(lax.axis_index("s") == 0)
    def _stage_tables():
        pltpu.sync_copy(t1_hbm, sh1)
        pltpu.sync_copy(t2_hbm, sh2)

    pltpu.sync_copy(i1_hbm.at[pl.ds(base, PER_W)], idx1)
    pltpu.sync_copy(i2_hbm.at[pl.ds(base, PER_W)], idx2)
    plsc.subcore_barrier()

    def gather_parts(table, idx, g, b):
        goff = (g * NBUF + b) * L
        return [
            (table.at[idx.at[pl.ds(goff, C1)]], rbuf.at[b, pl.ds(0, C1)]),
            (table.at[idx.at[pl.ds(goff + C1, C2)]], rbuf.at[b, pl.ds(C1, C2)]),
        ]

    def fire_gathers(table, idx, g, b, sem, add):
        for src, dst in gather_parts(table, idx, g, b):
            pltpu.async_copy(src, dst, sem.at[b], add=add)

    def wait_gathers(table, idx, g, b, sem):
        for src, dst in gather_parts(table, idx, g, b):
            pltpu.make_async_copy(src, dst, sem.at[b]).wait()

    def store_pair(g, b):
        return (rbuf.at[b],
                out_hbm.at[wid * BPW + g * NBUF + b, :, pl.ds(0, D)])

    def group(g, carry):
        for b in range(NBUF):
            @pl.when(g > 0)
            def _drain(b=b):
                src, dst = store_pair(g, b)
                pltpu.make_async_copy(src, dst, sem3.at[b]).wait()
            fire_gathers(sh1, idx1, g, b, sem1, False)
        for b in range(NBUF):
            wait_gathers(sh1, idx1, g, b, sem1)
            fire_gathers(sh2, idx2, g, b, sem2, True)
        for b in range(NBUF):
            wait_gathers(sh2, idx2, g, b, sem2)
            src, dst = store_pair(g, b)
            pltpu.async_copy(src, dst, sem3.at[b])
        return carry

    lax.fori_loop(0, NGRP, group, 0)

    for b in range(NBUF):
        src, dst = store_pair(NGRP - 1, b)
        pltpu.make_async_copy(src, dst, sem3.at[b]).wait()


def kernel(pos_1, pos_2, table1, table2):
    zrow = jnp.zeros((1, D), jnp.float32)
    t1a = jnp.concatenate([zrow, table1[: V - 1]], axis=0)
    t2a = jnp.concatenate([zrow, table2[: V - 1]], axis=0)
    i1 = pos_1.reshape(N).astype(jnp.int32)
    i2 = pos_2.reshape(N).astype(jnp.int32)
    out = _emb_sum_kernel(i1, i2, t1a, t2a)
    return out[..., :D]
